# baseline (speedup 1.0000x reference)
import functools

import jax
import jax.numpy as jnp
from jax import lax
from jax.experimental import pallas as pl
from jax.experimental.pallas import tpu as pltpu
from jax.experimental.pallas import tpu_sc as plsc

N = 10000
E = 320000
D_IN = 128
D_HID = 64
D_OUT = 7

NC = 2
NS = 16
NW = NC * NS
CHUNK = 128
CPT = 80
E_PAD = NW * CPT * CHUNK
N_PAD = 10112
RPT = N_PAD // NS

_MESH = plsc.VectorSubcoreMesh(core_axis_name="c", subcore_axis_name="s")


@functools.partial(
    pl.kernel,
    out_type=jax.ShapeDtypeStruct((NC, N_PAD, 16), jnp.float32),
    mesh=_MESH,
    scratch_types=[
        pltpu.VMEM((CPT, CHUNK), jnp.int32),
        pltpu.VMEM((CHUNK, 16), jnp.float32),
        pltpu.VMEM((RPT, 16), jnp.float32),
        pltpu.VMEM_SHARED((N_PAD, 16), jnp.float32),
        pltpu.SemaphoreType.DMA,
    ],
    compiler_params=pltpu.CompilerParams(use_tc_tiling_on_sc=False),
)
def _deg_kernel(dst_hbm, out_hbm, didx, ones, buf, acc, sem):
    c = lax.axis_index("c")
    s = lax.axis_index("s")
    wid = s * NC + c

    def _ones(i, carry):
        ones[i, :] = jnp.full((16,), 1.0, jnp.float32)
        return carry

    lax.fori_loop(0, CHUNK, _ones, 0)

    def _zero(i, carry):
        buf[i, :] = jnp.zeros((16,), jnp.float32)
        return carry

    lax.fori_loop(0, RPT, _zero, 0)

    pltpu.sync_copy(dst_hbm.at[pl.ds(wid * CPT, CPT)], didx)
    pltpu.sync_copy(buf, acc.at[pl.ds(s * RPT, RPT)])
    plsc.subcore_barrier()

    def _body(j, carry):
        pltpu.async_copy(ones, acc.at[didx.at[j]], sem, add=True)
        return carry

    lax.fori_loop(0, CPT, _body, 0)

    def _drain(j, carry):
        pltpu.make_async_copy(ones, acc.at[didx.at[0]], sem).wait()
        return carry

    lax.fori_loop(0, CPT, _drain, 0)
    plsc.subcore_barrier()

    pltpu.sync_copy(acc.at[pl.ds(s * RPT, RPT)], buf)
    pltpu.sync_copy(buf, out_hbm.at[c, pl.ds(s * RPT, RPT)])


def _edge_pipeline(sidx, didx, hs_sp, acc, rows, gsems, ssems):
    for k in range(4):
        pltpu.async_copy(hs_sp.at[sidx.at[k]], rows[k], gsems[k])

    def _body(t, carry):
        for k in range(4):
            j = 4 * t + k
            pltpu.make_async_copy(hs_sp.at[sidx.at[j]], rows[k], gsems[k]).wait()
            pltpu.async_copy(rows[k], acc.at[didx.at[j]], ssems[k], add=True)
        for k in range(4):
            j = 4 * t + k

            @pl.when(j + 4 < CPT)
            def _():
                pltpu.make_async_copy(rows[k], acc.at[didx.at[0]], ssems[k]).wait()
                pltpu.async_copy(hs_sp.at[sidx.at[j + 4]], rows[k], gsems[k])

        return carry

    lax.fori_loop(0, CPT // 4, _body, 0)
    for k in range(4):
        pltpu.make_async_copy(rows[k], acc.at[didx.at[0]], ssems[k]).wait()


@functools.partial(
    pl.kernel,
    out_type=[
        jax.ShapeDtypeStruct((NC, N_PAD, 32), jnp.float32),
        jax.ShapeDtypeStruct((NC, N_PAD, 32), jnp.float32),
    ],
    mesh=_MESH,
    scratch_types=[
        pltpu.VMEM((CPT, CHUNK), jnp.int32),
        pltpu.VMEM((CPT, CHUNK), jnp.int32),
        [pltpu.VMEM((CHUNK, 32), jnp.float32)] * 4,
        pltpu.VMEM((RPT, 32), jnp.float32),
        pltpu.VMEM_SHARED((N_PAD, 32), jnp.float32),
        pltpu.VMEM_SHARED((N, 32), jnp.float32),
        [pltpu.SemaphoreType.DMA] * 4,
        [pltpu.SemaphoreType.DMA] * 4,
    ],
    compiler_params=pltpu.CompilerParams(use_tc_tiling_on_sc=False),
)
def _agg32x2(src_hbm, dst_hbm, hsa_hbm, hsb_hbm, outa_hbm, outb_hbm,
             sidx, didx, rows, buf, acc, hs_sp, gsems, ssems):
    c = lax.axis_index("c")
    s = lax.axis_index("s")
    wid = s * NC + c

    pltpu.sync_copy(src_hbm.at[pl.ds(wid * CPT, CPT)], sidx)
    pltpu.sync_copy(dst_hbm.at[pl.ds(wid * CPT, CPT)], didx)

    for hs_hbm, out_hbm in ((hsa_hbm, outa_hbm), (hsb_hbm, outb_hbm)):

        def _zero(i, carry):
            for k in range(2):
                buf[i, pl.ds(k * 16, 16)] = jnp.zeros((16,), jnp.float32)
            return carry

        lax.fori_loop(0, RPT, _zero, 0)
        pltpu.sync_copy(buf, acc.at[pl.ds(s * RPT, RPT)])
        pltpu.sync_copy(hs_hbm.at[pl.ds(s * 625, 625)], buf.at[pl.ds(0, 625)])
        pltpu.sync_copy(buf.at[pl.ds(0, 625)], hs_sp.at[pl.ds(s * 625, 625)])
        plsc.subcore_barrier()

        _edge_pipeline(sidx, didx, hs_sp, acc, rows, gsems, ssems)
        plsc.subcore_barrier()

        pltpu.sync_copy(acc.at[pl.ds(s * RPT, RPT)], buf)
        pltpu.sync_copy(buf, out_hbm.at[c, pl.ds(s * RPT, RPT)])


def _make_agg(width):
    @functools.partial(
        pl.kernel,
        out_type=jax.ShapeDtypeStruct((NC, N_PAD, width), jnp.float32),
        mesh=_MESH,
        scratch_types=[
            pltpu.VMEM((CPT, CHUNK), jnp.int32),
            pltpu.VMEM((CPT, CHUNK), jnp.int32),
            [pltpu.VMEM((CHUNK, width), jnp.float32)] * 4,
            pltpu.VMEM((RPT, width), jnp.float32),
            pltpu.VMEM_SHARED((N_PAD, width), jnp.float32),
            pltpu.VMEM_SHARED((N, width), jnp.float32),
            [pltpu.SemaphoreType.DMA] * 4,
            [pltpu.SemaphoreType.DMA] * 4,
        ],
        compiler_params=pltpu.CompilerParams(use_tc_tiling_on_sc=False),
    )
    def _agg(src_hbm, dst_hbm, hs_hbm, out_hbm, sidx, didx, rows,
             buf, acc, hs_sp, gsems, ssems):
        c = lax.axis_index("c")
        s = lax.axis_index("s")
        wid = s * NC + c

        def _zero(i, carry):
            for k in range(width // 16):
                buf[i, pl.ds(k * 16, 16)] = jnp.zeros((16,), jnp.float32)
            return carry

        lax.fori_loop(0, RPT, _zero, 0)

        pltpu.sync_copy(src_hbm.at[pl.ds(wid * CPT, CPT)], sidx)
        pltpu.sync_copy(dst_hbm.at[pl.ds(wid * CPT, CPT)], didx)
        pltpu.sync_copy(buf, acc.at[pl.ds(s * RPT, RPT)])
        pltpu.sync_copy(hs_hbm.at[pl.ds(s * 625, 625)], buf.at[pl.ds(0, 625)])
        pltpu.sync_copy(buf.at[pl.ds(0, 625)], hs_sp.at[pl.ds(s * 625, 625)])
        plsc.subcore_barrier()

        _edge_pipeline(sidx, didx, hs_sp, acc, rows, gsems, ssems)
        plsc.subcore_barrier()

        pltpu.sync_copy(acc.at[pl.ds(s * RPT, RPT)], buf)
        pltpu.sync_copy(buf, out_hbm.at[c, pl.ds(s * RPT, RPT)])

    return _agg


_agg16 = _make_agg(16)


_R = 2000


def _tca_body(degp_ref, x_ref, w1_ref, hs1a_ref, hs1b_ref, dinv_ref):
    deg = degp_ref[0] + degp_ref[1] + 1.0
    dinv = lax.rsqrt(deg)
    h = jnp.dot(x_ref[...], w1_ref[...], preferred_element_type=jnp.float32)
    hs = h * dinv[:, :1]
    hs1a_ref[...] = hs[:, :32]
    hs1b_ref[...] = hs[:, 32:]
    dinv_ref[...] = dinv


def _tcb_body(acca_ref, accb_ref, hs1a_ref, hs1b_ref, dinv_ref, w2_ref,
              b1_ref, hs2_ref):
    acca = acca_ref[0] + acca_ref[1] + hs1a_ref[...]
    accb = accb_ref[0] + accb_ref[1] + hs1b_ref[...]
    acc = jnp.concatenate([acca, accb], axis=1)
    out1 = acc * dinv_ref[:, :1] + b1_ref[...]
    h2 = jnp.dot(out1, w2_ref[...], preferred_element_type=jnp.float32)
    hs2_ref[...] = h2 * dinv_ref[:, :1]


def _tcc_body(accp_ref, hs2_ref, dinv_ref, b2_ref, out_ref):
    acc = accp_ref[0] + accp_ref[1] + hs2_ref[...]
    res = acc * dinv_ref[:, :1] + b2_ref[...]
    out_ref[...] = res[:, :D_OUT]


def _rows_spec(width):
    return pl.BlockSpec((_R, width), lambda i: (i, 0))


def _part_spec(width):
    return pl.BlockSpec((NC, _R, width), lambda i: (0, i, 0))


def _full_spec(a, b):
    return pl.BlockSpec((a, b), lambda i: (0, 0))


_GRID = N // _R

_tca = pl.pallas_call(
    _tca_body,
    grid=(_GRID,),
    in_specs=[_part_spec(16), _rows_spec(D_IN), _full_spec(D_IN, D_HID)],
    out_specs=[_rows_spec(32), _rows_spec(32), _rows_spec(16)],
    out_shape=[
        jax.ShapeDtypeStruct((N, 32), jnp.float32),
        jax.ShapeDtypeStruct((N, 32), jnp.float32),
        jax.ShapeDtypeStruct((N, 16), jnp.float32),
    ],
)

_tcb = pl.pallas_call(
    _tcb_body,
    grid=(_GRID,),
    in_specs=[
        _part_spec(32),
        _part_spec(32),
        _rows_spec(32),
        _rows_spec(32),
        _rows_spec(16),
        _full_spec(D_HID, 16),
        _full_spec(1, D_HID),
    ],
    out_specs=[_rows_spec(16)],
    out_shape=[jax.ShapeDtypeStruct((N, 16), jnp.float32)],
)

_tcc = pl.pallas_call(
    _tcc_body,
    grid=(_GRID,),
    in_specs=[
        _part_spec(16),
        _rows_spec(16),
        _rows_spec(16),
        _full_spec(1, 16),
    ],
    out_specs=[_rows_spec(D_OUT)],
    out_shape=[jax.ShapeDtypeStruct((N, D_OUT), jnp.float32)],
)


def kernel(x, edge_index, W1, b1, W2, b2):
    pad = E_PAD - E
    src = jnp.concatenate(
        [edge_index[0], jnp.zeros((pad,), jnp.int32)]
    ).reshape(E_PAD // CHUNK, CHUNK)
    dst = jnp.concatenate(
        [edge_index[1], jnp.full((pad,), N, jnp.int32)]
    ).reshape(E_PAD // CHUNK, CHUNK)

    w2p = jnp.pad(W2, ((0, 0), (0, 16 - D_OUT)))
    b1r = b1.reshape(1, D_HID)
    b2r = jnp.pad(b2, (0, 16 - D_OUT)).reshape(1, 16)

    degp = _deg_kernel(dst)
    hs1a, hs1b, dinv = _tca(degp, x, W1)
    acc1a, acc1b = _agg32x2(src, dst, hs1a, hs1b)
    (hs2,) = _tcb(acc1a, acc1b, hs1a, hs1b, dinv, w2p, b1r)
    acc2 = _agg16(src, dst, hs2)
    (out,) = _tcc(acc2, hs2, dinv, b2r)
    return (out, 0)


# speedup vs baseline: 1.0712x; 1.0712x over previous
import functools

import jax
import jax.numpy as jnp
from jax import lax
from jax.experimental import pallas as pl
from jax.experimental.pallas import tpu as pltpu
from jax.experimental.pallas import tpu_sc as plsc

N = 10000
E = 320000
D_IN = 128
D_HID = 64
D_OUT = 7

NC = 2
NS = 16
NW = NC * NS
CHUNK = 128
CPT = 80
E_PAD = NW * CPT * CHUNK
N_PAD = 10112
RPT = N_PAD // NS

_MESH = plsc.VectorSubcoreMesh(core_axis_name="c", subcore_axis_name="s")


@functools.partial(
    pl.kernel,
    out_type=jax.ShapeDtypeStruct((NC, N_PAD, 16), jnp.float32),
    mesh=_MESH,
    scratch_types=[
        pltpu.VMEM((CPT, CHUNK), jnp.int32),
        pltpu.VMEM((CHUNK, 16), jnp.float32),
        pltpu.VMEM((RPT, 16), jnp.float32),
        pltpu.VMEM_SHARED((N_PAD, 16), jnp.float32),
        pltpu.SemaphoreType.DMA,
    ],
    compiler_params=pltpu.CompilerParams(use_tc_tiling_on_sc=False),
)
def _deg_kernel(dst_hbm, out_hbm, didx, ones, buf, acc, sem):
    c = lax.axis_index("c")
    s = lax.axis_index("s")
    wid = s * NC + c

    def _ones(i, carry):
        ones[i, :] = jnp.full((16,), 1.0, jnp.float32)
        return carry

    lax.fori_loop(0, CHUNK, _ones, 0)

    def _zero(i, carry):
        buf[i, :] = jnp.zeros((16,), jnp.float32)
        return carry

    lax.fori_loop(0, RPT, _zero, 0)

    pltpu.sync_copy(dst_hbm.at[pl.ds(wid * CPT, CPT)], didx)
    pltpu.sync_copy(buf, acc.at[pl.ds(s * RPT, RPT)])
    plsc.subcore_barrier()

    def _body(j, carry):
        pltpu.async_copy(ones, acc.at[didx.at[j]], sem, add=True)
        return carry

    lax.fori_loop(0, CPT, _body, 0)

    def _drain(j, carry):
        pltpu.make_async_copy(ones, acc.at[didx.at[0]], sem).wait()
        return carry

    lax.fori_loop(0, CPT, _drain, 0)
    plsc.subcore_barrier()

    pltpu.sync_copy(acc.at[pl.ds(s * RPT, RPT)], buf)
    pltpu.sync_copy(buf, out_hbm.at[c, pl.ds(s * RPT, RPT)])


def _edge_pipeline(sidx, didx, hs_sp, acc, rows, gsems, ssems):
    del ssems
    rows0, rows1 = rows[0], rows[1]
    sem0, sem1 = gsems[0], gsems[1]
    pltpu.async_copy(hs_sp.at[sidx.at[0]], rows0, sem0)

    def _body(t, carry):
        a = 2 * t
        pltpu.async_copy(hs_sp.at[sidx.at[a + 1]], rows1, sem1)
        pltpu.make_async_copy(hs_sp.at[sidx.at[a]], rows0, sem0).wait()
        pltpu.sync_copy(rows0, acc.at[didx.at[a]], add=True)

        @pl.when(a + 2 < CPT)
        def _():
            pltpu.async_copy(hs_sp.at[sidx.at[a + 2]], rows0, sem0)

        pltpu.make_async_copy(hs_sp.at[sidx.at[a + 1]], rows1, sem1).wait()
        pltpu.sync_copy(rows1, acc.at[didx.at[a + 1]], add=True)
        return carry

    lax.fori_loop(0, CPT // 2, _body, 0)


@functools.partial(
    pl.kernel,
    out_type=[
        jax.ShapeDtypeStruct((NC, N_PAD, 32), jnp.float32),
        jax.ShapeDtypeStruct((NC, N_PAD, 32), jnp.float32),
    ],
    mesh=_MESH,
    scratch_types=[
        pltpu.VMEM((CPT, CHUNK), jnp.int32),
        pltpu.VMEM((CPT, CHUNK), jnp.int32),
        [pltpu.VMEM((CHUNK, 32), jnp.float32)] * 4,
        pltpu.VMEM((RPT, 32), jnp.float32),
        pltpu.VMEM_SHARED((N_PAD, 32), jnp.float32),
        pltpu.VMEM_SHARED((N, 32), jnp.float32),
        [pltpu.SemaphoreType.DMA] * 4,
        [pltpu.SemaphoreType.DMA] * 4,
    ],
    compiler_params=pltpu.CompilerParams(use_tc_tiling_on_sc=False),
)
def _agg32x2(src_hbm, dst_hbm, hsa_hbm, hsb_hbm, outa_hbm, outb_hbm,
             sidx, didx, rows, buf, acc, hs_sp, gsems, ssems):
    c = lax.axis_index("c")
    s = lax.axis_index("s")
    wid = s * NC + c

    pltpu.sync_copy(src_hbm.at[pl.ds(wid * CPT, CPT)], sidx)
    pltpu.sync_copy(dst_hbm.at[pl.ds(wid * CPT, CPT)], didx)

    for hs_hbm, out_hbm in ((hsa_hbm, outa_hbm), (hsb_hbm, outb_hbm)):

        def _zero(i, carry):
            for k in range(2):
                buf[i, pl.ds(k * 16, 16)] = jnp.zeros((16,), jnp.float32)
            return carry

        lax.fori_loop(0, RPT, _zero, 0)
        pltpu.sync_copy(buf, acc.at[pl.ds(s * RPT, RPT)])
        pltpu.sync_copy(hs_hbm.at[pl.ds(s * 625, 625)], buf.at[pl.ds(0, 625)])
        pltpu.sync_copy(buf.at[pl.ds(0, 625)], hs_sp.at[pl.ds(s * 625, 625)])
        plsc.subcore_barrier()

        _edge_pipeline(sidx, didx, hs_sp, acc, rows, gsems, ssems)
        plsc.subcore_barrier()

        pltpu.sync_copy(acc.at[pl.ds(s * RPT, RPT)], buf)
        pltpu.sync_copy(buf, out_hbm.at[c, pl.ds(s * RPT, RPT)])


def _make_agg(width):
    @functools.partial(
        pl.kernel,
        out_type=jax.ShapeDtypeStruct((NC, N_PAD, width), jnp.float32),
        mesh=_MESH,
        scratch_types=[
            pltpu.VMEM((CPT, CHUNK), jnp.int32),
            pltpu.VMEM((CPT, CHUNK), jnp.int32),
            [pltpu.VMEM((CHUNK, width), jnp.float32)] * 4,
            pltpu.VMEM((RPT, width), jnp.float32),
            pltpu.VMEM_SHARED((N_PAD, width), jnp.float32),
            pltpu.VMEM_SHARED((N, width), jnp.float32),
            [pltpu.SemaphoreType.DMA] * 4,
            [pltpu.SemaphoreType.DMA] * 4,
        ],
        compiler_params=pltpu.CompilerParams(use_tc_tiling_on_sc=False),
    )
    def _agg(src_hbm, dst_hbm, hs_hbm, out_hbm, sidx, didx, rows,
             buf, acc, hs_sp, gsems, ssems):
        c = lax.axis_index("c")
        s = lax.axis_index("s")
        wid = s * NC + c

        def _zero(i, carry):
            for k in range(width // 16):
                buf[i, pl.ds(k * 16, 16)] = jnp.zeros((16,), jnp.float32)
            return carry

        lax.fori_loop(0, RPT, _zero, 0)

        pltpu.sync_copy(src_hbm.at[pl.ds(wid * CPT, CPT)], sidx)
        pltpu.sync_copy(dst_hbm.at[pl.ds(wid * CPT, CPT)], didx)
        pltpu.sync_copy(buf, acc.at[pl.ds(s * RPT, RPT)])
        pltpu.sync_copy(hs_hbm.at[pl.ds(s * 625, 625)], buf.at[pl.ds(0, 625)])
        pltpu.sync_copy(buf.at[pl.ds(0, 625)], hs_sp.at[pl.ds(s * 625, 625)])
        plsc.subcore_barrier()

        _edge_pipeline(sidx, didx, hs_sp, acc, rows, gsems, ssems)
        plsc.subcore_barrier()

        pltpu.sync_copy(acc.at[pl.ds(s * RPT, RPT)], buf)
        pltpu.sync_copy(buf, out_hbm.at[c, pl.ds(s * RPT, RPT)])

    return _agg


_agg16 = _make_agg(16)


_R = 2000


def _tca_body(degp_ref, x_ref, w1_ref, hs1a_ref, hs1b_ref, dinv_ref):
    deg = degp_ref[0] + degp_ref[1] + 1.0
    dinv = lax.rsqrt(deg)
    h = jnp.dot(x_ref[...], w1_ref[...], preferred_element_type=jnp.float32)
    hs = h * dinv[:, :1]
    hs1a_ref[...] = hs[:, :32]
    hs1b_ref[...] = hs[:, 32:]
    dinv_ref[...] = dinv


def _tcb_body(acca_ref, accb_ref, hs1a_ref, hs1b_ref, dinv_ref, w2_ref,
              b1_ref, hs2_ref):
    acca = acca_ref[0] + acca_ref[1] + hs1a_ref[...]
    accb = accb_ref[0] + accb_ref[1] + hs1b_ref[...]
    acc = jnp.concatenate([acca, accb], axis=1)
    out1 = acc * dinv_ref[:, :1] + b1_ref[...]
    h2 = jnp.dot(out1, w2_ref[...], preferred_element_type=jnp.float32)
    hs2_ref[...] = h2 * dinv_ref[:, :1]


def _tcc_body(accp_ref, hs2_ref, dinv_ref, b2_ref, out_ref):
    acc = accp_ref[0] + accp_ref[1] + hs2_ref[...]
    res = acc * dinv_ref[:, :1] + b2_ref[...]
    out_ref[...] = res[:, :D_OUT]


def _rows_spec(width):
    return pl.BlockSpec((_R, width), lambda i: (i, 0))


def _part_spec(width):
    return pl.BlockSpec((NC, _R, width), lambda i: (0, i, 0))


def _full_spec(a, b):
    return pl.BlockSpec((a, b), lambda i: (0, 0))


_GRID = N // _R

_tca = pl.pallas_call(
    _tca_body,
    grid=(_GRID,),
    in_specs=[_part_spec(16), _rows_spec(D_IN), _full_spec(D_IN, D_HID)],
    out_specs=[_rows_spec(32), _rows_spec(32), _rows_spec(16)],
    out_shape=[
        jax.ShapeDtypeStruct((N, 32), jnp.float32),
        jax.ShapeDtypeStruct((N, 32), jnp.float32),
        jax.ShapeDtypeStruct((N, 16), jnp.float32),
    ],
)

_tcb = pl.pallas_call(
    _tcb_body,
    grid=(_GRID,),
    in_specs=[
        _part_spec(32),
        _part_spec(32),
        _rows_spec(32),
        _rows_spec(32),
        _rows_spec(16),
        _full_spec(D_HID, 16),
        _full_spec(1, D_HID),
    ],
    out_specs=[_rows_spec(16)],
    out_shape=[jax.ShapeDtypeStruct((N, 16), jnp.float32)],
)

_tcc = pl.pallas_call(
    _tcc_body,
    grid=(_GRID,),
    in_specs=[
        _part_spec(16),
        _rows_spec(16),
        _rows_spec(16),
        _full_spec(1, 16),
    ],
    out_specs=[_rows_spec(D_OUT)],
    out_shape=[jax.ShapeDtypeStruct((N, D_OUT), jnp.float32)],
)


def kernel(x, edge_index, W1, b1, W2, b2):
    pad = E_PAD - E
    src = jnp.concatenate(
        [edge_index[0], jnp.zeros((pad,), jnp.int32)]
    ).reshape(E_PAD // CHUNK, CHUNK)
    dst = jnp.concatenate(
        [edge_index[1], jnp.full((pad,), N, jnp.int32)]
    ).reshape(E_PAD // CHUNK, CHUNK)

    w2p = jnp.pad(W2, ((0, 0), (0, 16 - D_OUT)))
    b1r = b1.reshape(1, D_HID)
    b2r = jnp.pad(b2, (0, 16 - D_OUT)).reshape(1, 16)

    degp = _deg_kernel(dst)
    hs1a, hs1b, dinv = _tca(degp, x, W1)
    acc1a, acc1b = _agg32x2(src, dst, hs1a, hs1b)
    (hs2,) = _tcb(acc1a, acc1b, hs1a, hs1b, dinv, w2p, b1r)
    acc2 = _agg16(src, dst, hs2)
    (out,) = _tcc(acc2, hs2, dinv, b2r)
    return (out, 0)


# speedup vs baseline: 1.1866x; 1.1078x over previous
import functools

import jax
import jax.numpy as jnp
from jax import lax
from jax.experimental import pallas as pl
from jax.experimental.pallas import tpu as pltpu
from jax.experimental.pallas import tpu_sc as plsc

N = 10000
E = 320000
D_IN = 128
D_HID = 64
D_OUT = 7

NC = 2
NS = 16
NW = NC * NS
CHUNK = 128
NCH = E // CHUNK
CPT = 79
N_PAD = 10112
RPT = N_PAD // NS

_MESH = plsc.VectorSubcoreMesh(core_axis_name="c", subcore_axis_name="s")


def _chunk_range(wid):
    base = wid * 78 + jnp.minimum(wid, 4)
    nc = 78 + jnp.where(wid < 4, 1, 0)
    return base, nc


def _load_idx(e_hbm, row, base, wid, dest):
    pltpu.sync_copy(e_hbm.at[row, pl.ds(base, 78)], dest.at[pl.ds(0, 78)])

    @pl.when(wid < 4)
    def _():
        pltpu.sync_copy(e_hbm.at[row, pl.ds(base + 78, 1)],
                        dest.at[pl.ds(78, 1)])


@functools.partial(
    pl.kernel,
    out_type=jax.ShapeDtypeStruct((NC, N_PAD, 16), jnp.float32),
    mesh=_MESH,
    scratch_types=[
        pltpu.VMEM((CPT, CHUNK), jnp.int32),
        pltpu.VMEM((CHUNK, 16), jnp.float32),
        pltpu.VMEM((RPT, 16), jnp.float32),
        pltpu.VMEM_SHARED((N_PAD, 16), jnp.float32),
        pltpu.SemaphoreType.DMA,
    ],
    compiler_params=pltpu.CompilerParams(use_tc_tiling_on_sc=False),
)
def _deg_kernel(e_hbm, out_hbm, didx, ones, buf, acc, sem):
    c = lax.axis_index("c")
    s = lax.axis_index("s")
    wid = s * NC + c
    base, nc = _chunk_range(wid)

    def _ones(i, carry):
        ones[i, :] = jnp.full((16,), 1.0, jnp.float32)
        return carry

    lax.fori_loop(0, CHUNK, _ones, 0)

    def _zero(i, carry):
        buf[i, :] = jnp.zeros((16,), jnp.float32)
        return carry

    lax.fori_loop(0, RPT, _zero, 0)

    _load_idx(e_hbm, 1, base, wid, didx)
    pltpu.sync_copy(buf, acc.at[pl.ds(s * RPT, RPT)])
    plsc.subcore_barrier()

    def _body(j, carry):
        pltpu.async_copy(ones, acc.at[didx.at[j]], sem, add=True)
        return carry

    lax.fori_loop(0, nc, _body, 0)

    def _drain(j, carry):
        pltpu.make_async_copy(ones, acc.at[didx.at[0]], sem).wait()
        return carry

    lax.fori_loop(0, nc, _drain, 0)
    plsc.subcore_barrier()

    pltpu.sync_copy(acc.at[pl.ds(s * RPT, RPT)], buf)
    pltpu.sync_copy(buf, out_hbm.at[c, pl.ds(s * RPT, RPT)])


def _edge_pipeline(nc, sidx, didx, hs_sp, acc, rows, gsems, ssems):
    del ssems
    rows0, rows1 = rows[0], rows[1]
    sem0, sem1 = gsems[0], gsems[1]
    pltpu.async_copy(hs_sp.at[sidx.at[0]], rows0, sem0)

    def _body(t, carry):
        a = 2 * t
        pltpu.async_copy(hs_sp.at[sidx.at[a + 1]], rows1, sem1)
        pltpu.make_async_copy(hs_sp.at[sidx.at[a]], rows0, sem0).wait()
        pltpu.sync_copy(rows0, acc.at[didx.at[a]], add=True)

        @pl.when(a + 2 < nc)
        def _():
            pltpu.async_copy(hs_sp.at[sidx.at[a + 2]], rows0, sem0)

        pltpu.make_async_copy(hs_sp.at[sidx.at[a + 1]], rows1, sem1).wait()
        pltpu.sync_copy(rows1, acc.at[didx.at[a + 1]], add=True)
        return carry

    lax.fori_loop(0, 39, _body, 0)

    @pl.when(nc > 78)
    def _():
        pltpu.make_async_copy(hs_sp.at[sidx.at[78]], rows0, sem0).wait()
        pltpu.sync_copy(rows0, acc.at[didx.at[78]], add=True)


@functools.partial(
    pl.kernel,
    out_type=[
        jax.ShapeDtypeStruct((NC, N_PAD, 32), jnp.float32),
        jax.ShapeDtypeStruct((NC, N_PAD, 32), jnp.float32),
    ],
    mesh=_MESH,
    scratch_types=[
        pltpu.VMEM((CPT, CHUNK), jnp.int32),
        pltpu.VMEM((CPT, CHUNK), jnp.int32),
        [pltpu.VMEM((CHUNK, 32), jnp.float32)] * 4,
        pltpu.VMEM((RPT, 32), jnp.float32),
        pltpu.VMEM_SHARED((N_PAD, 32), jnp.float32),
        pltpu.VMEM_SHARED((N, 32), jnp.float32),
        [pltpu.SemaphoreType.DMA] * 4,
        [pltpu.SemaphoreType.DMA] * 4,
    ],
    compiler_params=pltpu.CompilerParams(use_tc_tiling_on_sc=False),
)
def _agg32x2(e_hbm, hsa_hbm, hsb_hbm, outa_hbm, outb_hbm,
             sidx, didx, rows, buf, acc, hs_sp, gsems, ssems):
    c = lax.axis_index("c")
    s = lax.axis_index("s")
    wid = s * NC + c
    base, nc = _chunk_range(wid)

    _load_idx(e_hbm, 0, base, wid, sidx)
    _load_idx(e_hbm, 1, base, wid, didx)

    for hs_hbm, out_hbm in ((hsa_hbm, outa_hbm), (hsb_hbm, outb_hbm)):

        def _zero(i, carry):
            for k in range(2):
                buf[i, pl.ds(k * 16, 16)] = jnp.zeros((16,), jnp.float32)
            return carry

        lax.fori_loop(0, RPT, _zero, 0)
        pltpu.sync_copy(buf, acc.at[pl.ds(s * RPT, RPT)])
        pltpu.sync_copy(hs_hbm.at[pl.ds(s * 625, 625)], buf.at[pl.ds(0, 625)])
        pltpu.sync_copy(buf.at[pl.ds(0, 625)], hs_sp.at[pl.ds(s * 625, 625)])
        plsc.subcore_barrier()

        _edge_pipeline(nc, sidx, didx, hs_sp, acc, rows, gsems, ssems)
        plsc.subcore_barrier()

        pltpu.sync_copy(acc.at[pl.ds(s * RPT, RPT)], buf)
        pltpu.sync_copy(buf, out_hbm.at[c, pl.ds(s * RPT, RPT)])


def _make_agg(width):
    @functools.partial(
        pl.kernel,
        out_type=jax.ShapeDtypeStruct((NC, N_PAD, width), jnp.float32),
        mesh=_MESH,
        scratch_types=[
            pltpu.VMEM((CPT, CHUNK), jnp.int32),
            pltpu.VMEM((CPT, CHUNK), jnp.int32),
            [pltpu.VMEM((CHUNK, width), jnp.float32)] * 4,
            pltpu.VMEM((RPT, width), jnp.float32),
            pltpu.VMEM_SHARED((N_PAD, width), jnp.float32),
            pltpu.VMEM_SHARED((N, width), jnp.float32),
            [pltpu.SemaphoreType.DMA] * 4,
            [pltpu.SemaphoreType.DMA] * 4,
        ],
        compiler_params=pltpu.CompilerParams(use_tc_tiling_on_sc=False),
    )
    def _agg(e_hbm, hs_hbm, out_hbm, sidx, didx, rows,
             buf, acc, hs_sp, gsems, ssems):
        c = lax.axis_index("c")
        s = lax.axis_index("s")
        wid = s * NC + c
        base, nc = _chunk_range(wid)

        def _zero(i, carry):
            for k in range(width // 16):
                buf[i, pl.ds(k * 16, 16)] = jnp.zeros((16,), jnp.float32)
            return carry

        lax.fori_loop(0, RPT, _zero, 0)

        _load_idx(e_hbm, 0, base, wid, sidx)
        _load_idx(e_hbm, 1, base, wid, didx)
        pltpu.sync_copy(buf, acc.at[pl.ds(s * RPT, RPT)])
        pltpu.sync_copy(hs_hbm.at[pl.ds(s * 625, 625)], buf.at[pl.ds(0, 625)])
        pltpu.sync_copy(buf.at[pl.ds(0, 625)], hs_sp.at[pl.ds(s * 625, 625)])
        plsc.subcore_barrier()

        _edge_pipeline(nc, sidx, didx, hs_sp, acc, rows, gsems, ssems)
        plsc.subcore_barrier()

        pltpu.sync_copy(acc.at[pl.ds(s * RPT, RPT)], buf)
        pltpu.sync_copy(buf, out_hbm.at[c, pl.ds(s * RPT, RPT)])

    return _agg


_agg16 = _make_agg(16)


_R = 2000


def _tca_body(degp_ref, x_ref, w1_ref, hs1a_ref, hs1b_ref, dinv_ref):
    deg = degp_ref[0] + degp_ref[1] + 1.0
    dinv = lax.rsqrt(deg)
    h = jnp.dot(x_ref[...], w1_ref[...], preferred_element_type=jnp.float32)
    hs = h * dinv[:, :1]
    hs1a_ref[...] = hs[:, :32]
    hs1b_ref[...] = hs[:, 32:]
    dinv_ref[...] = dinv


def _tcb_body(acca_ref, accb_ref, hs1a_ref, hs1b_ref, dinv_ref, w2_ref,
              b1_ref, hs2_ref):
    acca = acca_ref[0] + acca_ref[1] + hs1a_ref[...]
    accb = accb_ref[0] + accb_ref[1] + hs1b_ref[...]
    acc = jnp.concatenate([acca, accb], axis=1)
    out1 = acc * dinv_ref[:, :1] + b1_ref[...]
    h2 = jnp.dot(out1, w2_ref[...], preferred_element_type=jnp.float32)
    hs2 = h2 * dinv_ref[:, :1]
    pad = jnp.zeros((hs2.shape[0], 16 - D_OUT), jnp.float32)
    hs2_ref[...] = jnp.concatenate([hs2, pad], axis=1)


def _tcc_body(accp_ref, hs2_ref, dinv_ref, b2_ref, out_ref):
    acc = accp_ref[0] + accp_ref[1] + hs2_ref[...]
    res = acc * dinv_ref[:, :1]
    out_ref[...] = res[:, :D_OUT] + b2_ref[...]


def _rows_spec(width):
    return pl.BlockSpec((_R, width), lambda i: (i, 0))


def _part_spec(width):
    return pl.BlockSpec((NC, _R, width), lambda i: (0, i, 0))


def _full_spec(a, b):
    return pl.BlockSpec((a, b), lambda i: (0, 0))


_GRID = N // _R

_tca = pl.pallas_call(
    _tca_body,
    grid=(_GRID,),
    in_specs=[_part_spec(16), _rows_spec(D_IN), _full_spec(D_IN, D_HID)],
    out_specs=[_rows_spec(32), _rows_spec(32), _rows_spec(16)],
    out_shape=[
        jax.ShapeDtypeStruct((N, 32), jnp.float32),
        jax.ShapeDtypeStruct((N, 32), jnp.float32),
        jax.ShapeDtypeStruct((N, 16), jnp.float32),
    ],
)

_tcb = pl.pallas_call(
    _tcb_body,
    grid=(_GRID,),
    in_specs=[
        _part_spec(32),
        _part_spec(32),
        _rows_spec(32),
        _rows_spec(32),
        _rows_spec(16),
        _full_spec(D_HID, D_OUT),
        _full_spec(1, D_HID),
    ],
    out_specs=[_rows_spec(16)],
    out_shape=[jax.ShapeDtypeStruct((N, 16), jnp.float32)],
)

_tcc = pl.pallas_call(
    _tcc_body,
    grid=(_GRID,),
    in_specs=[
        _part_spec(16),
        _rows_spec(16),
        _rows_spec(16),
        _full_spec(1, D_OUT),
    ],
    out_specs=[_rows_spec(D_OUT)],
    out_shape=[jax.ShapeDtypeStruct((N, D_OUT), jnp.float32)],
)


def kernel(x, edge_index, W1, b1, W2, b2):
    e3 = edge_index.reshape(2, NCH, CHUNK)
    b1r = b1.reshape(1, D_HID)
    b2r = b2.reshape(1, D_OUT)

    degp = _deg_kernel(e3)
    hs1a, hs1b, dinv = _tca(degp, x, W1)
    acc1a, acc1b = _agg32x2(e3, hs1a, hs1b)
    (hs2,) = _tcb(acc1a, acc1b, hs1a, hs1b, dinv, W2, b1r)
    acc2 = _agg16(e3, hs2)
    (out,) = _tcc(acc2, hs2, dinv, b2r)
    return (out, 0)


# speedup vs baseline: 1.1880x; 1.0012x over previous
import functools

import jax
import jax.numpy as jnp
from jax import lax
from jax.experimental import pallas as pl
from jax.experimental.pallas import tpu as pltpu
from jax.experimental.pallas import tpu_sc as plsc

N = 10000
E = 320000
D_IN = 128
D_HID = 64
D_OUT = 7

NC = 2
NS = 16
NW = NC * NS
CHUNK = 128
NCH = E // CHUNK
CPT = 79
N_PAD = 10112
RPT = N_PAD // NS

_MESH = plsc.VectorSubcoreMesh(core_axis_name="c", subcore_axis_name="s")


def _chunk_range(wid):
    base = wid * 78 + jnp.minimum(wid, 4)
    nc = 78 + jnp.where(wid < 4, 1, 0)
    return base, nc


def _load_idx(e_hbm, row, base, wid, dest):
    pltpu.sync_copy(e_hbm.at[row, pl.ds(base, 78)], dest.at[pl.ds(0, 78)])

    @pl.when(wid < 4)
    def _():
        pltpu.sync_copy(e_hbm.at[row, pl.ds(base + 78, 1)],
                        dest.at[pl.ds(78, 1)])


@functools.partial(
    pl.kernel,
    out_type=jax.ShapeDtypeStruct((NC, N_PAD, 16), jnp.float32),
    mesh=_MESH,
    scratch_types=[
        pltpu.VMEM((CPT, CHUNK), jnp.int32),
        pltpu.VMEM((CHUNK, 16), jnp.float32),
        pltpu.VMEM((RPT, 16), jnp.float32),
        pltpu.VMEM_SHARED((N_PAD, 16), jnp.float32),
        pltpu.SemaphoreType.DMA,
    ],
    compiler_params=pltpu.CompilerParams(use_tc_tiling_on_sc=False),
)
def _deg_kernel(e_hbm, out_hbm, didx, ones, buf, acc, sem):
    c = lax.axis_index("c")
    s = lax.axis_index("s")
    wid = s * NC + c
    base, nc = _chunk_range(wid)

    def _ones(i, carry):
        ones[i, :] = jnp.full((16,), 1.0, jnp.float32)
        return carry

    lax.fori_loop(0, CHUNK, _ones, 0)

    def _zero(i, carry):
        buf[i, :] = jnp.zeros((16,), jnp.float32)
        return carry

    lax.fori_loop(0, RPT, _zero, 0)

    _load_idx(e_hbm, 1, base, wid, didx)
    pltpu.sync_copy(buf, acc.at[pl.ds(s * RPT, RPT)])
    plsc.subcore_barrier()

    def _body(j, carry):
        pltpu.async_copy(ones, acc.at[didx.at[j]], sem, add=True)
        return carry

    lax.fori_loop(0, nc, _body, 0)

    def _drain(j, carry):
        pltpu.make_async_copy(ones, acc.at[didx.at[0]], sem).wait()
        return carry

    lax.fori_loop(0, nc, _drain, 0)
    plsc.subcore_barrier()

    pltpu.sync_copy(acc.at[pl.ds(s * RPT, RPT)], buf)
    pltpu.sync_copy(buf, out_hbm.at[c, pl.ds(s * RPT, RPT)])


def _edge_pipeline(nc, sidx, didx, hs_sp, acc, rows, gsems, ssems):
    del ssems
    rows0, rows1 = rows[0], rows[1]
    sem0, sem1 = gsems[0], gsems[1]
    pltpu.async_copy(hs_sp.at[sidx.at[0]], rows0, sem0)

    def _body(t, carry):
        a = 2 * t
        pltpu.async_copy(hs_sp.at[sidx.at[a + 1]], rows1, sem1)
        pltpu.make_async_copy(hs_sp.at[sidx.at[a]], rows0, sem0).wait()
        pltpu.sync_copy(rows0, acc.at[didx.at[a]], add=True)

        @pl.when(a + 2 < nc)
        def _():
            pltpu.async_copy(hs_sp.at[sidx.at[a + 2]], rows0, sem0)

        pltpu.make_async_copy(hs_sp.at[sidx.at[a + 1]], rows1, sem1).wait()
        pltpu.sync_copy(rows1, acc.at[didx.at[a + 1]], add=True)
        return carry

    lax.fori_loop(0, 39, _body, 0)

    @pl.when(nc > 78)
    def _():
        pltpu.make_async_copy(hs_sp.at[sidx.at[78]], rows0, sem0).wait()
        pltpu.sync_copy(rows0, acc.at[didx.at[78]], add=True)


@functools.partial(
    pl.kernel,
    out_type=[
        jax.ShapeDtypeStruct((NC, N_PAD, 32), jnp.float32),
        jax.ShapeDtypeStruct((NC, N_PAD, 32), jnp.float32),
    ],
    mesh=_MESH,
    scratch_types=[
        pltpu.VMEM((CPT, CHUNK), jnp.int32),
        pltpu.VMEM((CPT, CHUNK), jnp.int32),
        [pltpu.VMEM((CHUNK, 32), jnp.float32)] * 4,
        pltpu.VMEM((RPT, 32), jnp.float32),
        pltpu.VMEM_SHARED((N_PAD, 32), jnp.float32),
        pltpu.VMEM_SHARED((N, 32), jnp.float32),
        [pltpu.SemaphoreType.DMA] * 4,
        [pltpu.SemaphoreType.DMA] * 4,
    ],
    compiler_params=pltpu.CompilerParams(use_tc_tiling_on_sc=False),
)
def _agg32x2(e_hbm, hsa_hbm, hsb_hbm, outa_hbm, outb_hbm,
             sidx, didx, rows, buf, acc, hs_sp, gsems, ssems):
    c = lax.axis_index("c")
    s = lax.axis_index("s")
    wid = s * NC + c
    base, nc = _chunk_range(wid)

    _load_idx(e_hbm, 0, base, wid, sidx)
    _load_idx(e_hbm, 1, base, wid, didx)

    for hs_hbm, out_hbm in ((hsa_hbm, outa_hbm), (hsb_hbm, outb_hbm)):

        def _zero(i, carry):
            for k in range(2):
                buf[i, pl.ds(k * 16, 16)] = jnp.zeros((16,), jnp.float32)
            return carry

        lax.fori_loop(0, RPT, _zero, 0)
        pltpu.sync_copy(buf, acc.at[pl.ds(s * RPT, RPT)])
        pltpu.sync_copy(hs_hbm.at[pl.ds(s * 625, 625)], buf.at[pl.ds(0, 625)])
        pltpu.sync_copy(buf.at[pl.ds(0, 625)], hs_sp.at[pl.ds(s * 625, 625)])
        plsc.subcore_barrier()

        _edge_pipeline(nc, sidx, didx, hs_sp, acc, rows, gsems, ssems)
        plsc.subcore_barrier()

        pltpu.sync_copy(acc.at[pl.ds(s * RPT, RPT)], buf)
        pltpu.sync_copy(buf, out_hbm.at[c, pl.ds(s * RPT, RPT)])


def _make_agg(width):
    @functools.partial(
        pl.kernel,
        out_type=jax.ShapeDtypeStruct((NC, N_PAD, width), jnp.float32),
        mesh=_MESH,
        scratch_types=[
            pltpu.VMEM((CPT, CHUNK), jnp.int32),
            pltpu.VMEM((CPT, CHUNK), jnp.int32),
            [pltpu.VMEM((CHUNK, width), jnp.float32)] * 4,
            pltpu.VMEM((RPT, width), jnp.float32),
            pltpu.VMEM_SHARED((N_PAD, width), jnp.float32),
            pltpu.VMEM_SHARED((N, width), jnp.float32),
            [pltpu.SemaphoreType.DMA] * 4,
            [pltpu.SemaphoreType.DMA] * 4,
        ],
        compiler_params=pltpu.CompilerParams(use_tc_tiling_on_sc=False),
    )
    def _agg(e_hbm, hs_hbm, out_hbm, sidx, didx, rows,
             buf, acc, hs_sp, gsems, ssems):
        c = lax.axis_index("c")
        s = lax.axis_index("s")
        wid = s * NC + c
        base, nc = _chunk_range(wid)

        def _zero(i, carry):
            for k in range(width // 16):
                buf[i, pl.ds(k * 16, 16)] = jnp.zeros((16,), jnp.float32)
            return carry

        lax.fori_loop(0, RPT, _zero, 0)

        _load_idx(e_hbm, 0, base, wid, sidx)
        _load_idx(e_hbm, 1, base, wid, didx)
        pltpu.sync_copy(buf, acc.at[pl.ds(s * RPT, RPT)])
        pltpu.sync_copy(hs_hbm.at[pl.ds(s * 625, 625)], buf.at[pl.ds(0, 625)])
        pltpu.sync_copy(buf.at[pl.ds(0, 625)], hs_sp.at[pl.ds(s * 625, 625)])
        plsc.subcore_barrier()

        _edge_pipeline(nc, sidx, didx, hs_sp, acc, rows, gsems, ssems)
        plsc.subcore_barrier()

        pltpu.sync_copy(acc.at[pl.ds(s * RPT, RPT)], buf)
        pltpu.sync_copy(buf, out_hbm.at[c, pl.ds(s * RPT, RPT)])

    return _agg


_agg16 = _make_agg(16)


_R = 2000


def _tcmm_body(x_ref, w1_ref, h_ref):
    h_ref[...] = jnp.dot(x_ref[...], w1_ref[...],
                         preferred_element_type=jnp.float32)


def _tca_body(degp_ref, h_ref, hs1a_ref, hs1b_ref, dinv_ref):
    deg = degp_ref[0] + degp_ref[1] + 1.0
    dinv = lax.rsqrt(deg)
    hs = h_ref[...] * dinv[:, :1]
    hs1a_ref[...] = hs[:, :32]
    hs1b_ref[...] = hs[:, 32:]
    dinv_ref[...] = dinv


def _tcb_body(acca_ref, accb_ref, hs1a_ref, hs1b_ref, dinv_ref, w2_ref,
              b1_ref, hs2_ref):
    acca = acca_ref[0] + acca_ref[1] + hs1a_ref[...]
    accb = accb_ref[0] + accb_ref[1] + hs1b_ref[...]
    acc = jnp.concatenate([acca, accb], axis=1)
    out1 = acc * dinv_ref[:, :1] + b1_ref[...]
    h2 = jnp.dot(out1, w2_ref[...], preferred_element_type=jnp.float32)
    hs2 = h2 * dinv_ref[:, :1]
    pad = jnp.zeros((hs2.shape[0], 16 - D_OUT), jnp.float32)
    hs2_ref[...] = jnp.concatenate([hs2, pad], axis=1)


def _tcc_body(accp_ref, hs2_ref, dinv_ref, b2_ref, out_ref):
    acc = accp_ref[0] + accp_ref[1] + hs2_ref[...]
    res = acc * dinv_ref[:, :1]
    out_ref[...] = res[:, :D_OUT] + b2_ref[...]


def _rows_spec(width):
    return pl.BlockSpec((_R, width), lambda i: (i, 0))


def _part_spec(width):
    return pl.BlockSpec((NC, _R, width), lambda i: (0, i, 0))


def _full_spec(a, b):
    return pl.BlockSpec((a, b), lambda i: (0, 0))


_GRID = N // _R

_tcmm = pl.pallas_call(
    _tcmm_body,
    grid=(_GRID,),
    in_specs=[_rows_spec(D_IN), _full_spec(D_IN, D_HID)],
    out_specs=[_rows_spec(D_HID)],
    out_shape=[jax.ShapeDtypeStruct((N, D_HID), jnp.float32)],
)

_tca = pl.pallas_call(
    _tca_body,
    grid=(_GRID,),
    in_specs=[_part_spec(16), _rows_spec(D_HID)],
    out_specs=[_rows_spec(32), _rows_spec(32), _rows_spec(16)],
    out_shape=[
        jax.ShapeDtypeStruct((N, 32), jnp.float32),
        jax.ShapeDtypeStruct((N, 32), jnp.float32),
        jax.ShapeDtypeStruct((N, 16), jnp.float32),
    ],
)

_tcb = pl.pallas_call(
    _tcb_body,
    grid=(_GRID,),
    in_specs=[
        _part_spec(32),
        _part_spec(32),
        _rows_spec(32),
        _rows_spec(32),
        _rows_spec(16),
        _full_spec(D_HID, D_OUT),
        _full_spec(1, D_HID),
    ],
    out_specs=[_rows_spec(16)],
    out_shape=[jax.ShapeDtypeStruct((N, 16), jnp.float32)],
)

_tcc = pl.pallas_call(
    _tcc_body,
    grid=(_GRID,),
    in_specs=[
        _part_spec(16),
        _rows_spec(16),
        _rows_spec(16),
        _full_spec(1, D_OUT),
    ],
    out_specs=[_rows_spec(D_OUT)],
    out_shape=[jax.ShapeDtypeStruct((N, D_OUT), jnp.float32)],
)


def kernel(x, edge_index, W1, b1, W2, b2):
    e3 = edge_index.reshape(2, NCH, CHUNK)
    b1r = b1.reshape(1, D_HID)
    b2r = b2.reshape(1, D_OUT)

    (h1,) = _tcmm(x, W1)
    degp = _deg_kernel(e3)
    hs1a, hs1b, dinv = _tca(degp, h1)
    acc1a, acc1b = _agg32x2(e3, hs1a, hs1b)
    (hs2,) = _tcb(acc1a, acc1b, hs1a, hs1b, dinv, W2, b1r)
    acc2 = _agg16(e3, hs2)
    (out,) = _tcc(acc2, hs2, dinv, b2r)
    return (out, 0)
